# baseline (speedup 1.0000x reference)
import jax
import jax.numpy as jnp
from jax.experimental import pallas as pl


_IPB = 2
_BQ = 256


def _make_cost_kernel(q, e, tq, ipb, bq):
    def _one_image(feat_ref, out_ref, b, i):
        es = pl.multiple_of(tq + b * e, e)
        tf = feat_ref[:, pl.ds(es, e)]
        tx0 = tf[0:1, :]
        ty0 = tf[1:2, :]
        tx1 = tf[2:3, :]
        ty1 = tf[3:4, :]
        tpx = tf[4:5, :]
        tpy = tf[5:6, :]
        area2 = (tx1 - tx0) * (ty1 - ty0)

        for s in range(0, q, bq):
            qs = pl.multiple_of(b * q + s, bq)
            pf = feat_ref[:, pl.ds(qs, bq)].T
            px0 = pf[:, 0:1]
            py0 = pf[:, 1:2]
            px1 = pf[:, 2:3]
            py1 = pf[:, 3:4]
            ppx = pf[:, 4:5]
            ppy = pf[:, 5:6]
            lg = pf[:, 6:7]

            area1 = (px1 - px0) * (py1 - py0)
            wx = jnp.maximum(
                jnp.minimum(px1, tx1) - jnp.maximum(px0, tx0), 0.0)
            wy = jnp.maximum(
                jnp.minimum(py1, ty1) - jnp.maximum(py0, ty0), 0.0)
            inter = wx * wy
            union = area1 + area2 - inter
            hull = (jnp.maximum(px1, tx1) - jnp.minimum(px0, tx0)) * (
                jnp.maximum(py1, ty1) - jnp.minimum(py0, ty0))
            qq = (inter * hull + union * union) / (union * hull)

            l1 = (jnp.abs(px0 - tx0) + jnp.abs(py0 - ty0)
                  + jnp.abs(px1 - tx1) + jnp.abs(py1 - ty1))

            dx = ppx - tpx
            dy = ppy - tpy
            sq = dx * dx + dy * dy

            z = -lg
            cls2 = 2.0 * (jnp.maximum(z, 0.0)
                          + jnp.log1p(jnp.exp(-jnp.abs(z)))) + 2.0

            out_ref[i, pl.ds(s, bq), :] = (cls2 + 5.0 * l1 - 2.0 * qq
                                           + 0.25 * sq)

    def _cost_kernel(feat_ref, out_ref):
        g = pl.program_id(0)
        for i in range(ipb):
            _one_image(feat_ref, out_ref, g * ipb + i, i)

    return _cost_kernel


def kernel(pred_logits, pred_boxes, pred_positions, true_boxes,
           true_positions, query_batch_offsets, electron_batch_offsets):
    nb = query_batch_offsets.shape[0] - 1
    tq = pred_logits.shape[0]
    te = true_boxes.shape[0]
    q = tq // nb
    e = te // nb
    zc = jnp.zeros((tq + te,), jnp.float32)
    feat = jnp.stack([
        jnp.concatenate([pred_boxes[:, 0], true_boxes[:, 0]]),
        jnp.concatenate([pred_boxes[:, 1], true_boxes[:, 1]]),
        jnp.concatenate([pred_boxes[:, 2], true_boxes[:, 2]]),
        jnp.concatenate([pred_boxes[:, 3], true_boxes[:, 3]]),
        jnp.concatenate([pred_positions[:, 0], true_positions[:, 0]]),
        jnp.concatenate([pred_positions[:, 1], true_positions[:, 1]]),
        jnp.concatenate([pred_logits, zc[:te]]),
        zc,
    ])
    return pl.pallas_call(
        _make_cost_kernel(q, e, tq, _IPB, _BQ),
        grid=(nb // _IPB,),
        in_specs=[pl.BlockSpec((8, tq + te), lambda b: (0, 0))],
        out_specs=pl.BlockSpec((_IPB, q, e), lambda b: (b, 0, 0)),
        out_shape=jax.ShapeDtypeStruct((nb, q, e), jnp.float32),
    )(feat)


# speedup vs baseline: 1.1501x; 1.1501x over previous
import jax
import jax.numpy as jnp
from jax.experimental import pallas as pl


_IPB = 2
_BQ = 256


def _make_cost_kernel(q, e, tq, ipb, bq):
    def _one_image(feat_ref, out_ref, b, i):
        es = pl.multiple_of(tq + b * e, e)
        tf = feat_ref[pl.ds(es, e), :].T
        tx0 = tf[0:1, :]
        ty0 = tf[1:2, :]
        tx1 = tf[2:3, :]
        ty1 = tf[3:4, :]
        tpx = tf[4:5, :]
        tpy = tf[5:6, :]
        area2 = (tx1 - tx0) * (ty1 - ty0)

        for s in range(0, q, bq):
            qs = pl.multiple_of(b * q + s, bq)
            pf = feat_ref[pl.ds(qs, bq), :]
            px0 = pf[:, 0:1]
            py0 = pf[:, 1:2]
            px1 = pf[:, 2:3]
            py1 = pf[:, 3:4]
            ppx = pf[:, 4:5]
            ppy = pf[:, 5:6]
            lg = pf[:, 6:7]

            area1 = (px1 - px0) * (py1 - py0)
            wx = jnp.maximum(
                jnp.minimum(px1, tx1) - jnp.maximum(px0, tx0), 0.0)
            wy = jnp.maximum(
                jnp.minimum(py1, ty1) - jnp.maximum(py0, ty0), 0.0)
            inter = wx * wy
            union = area1 + area2 - inter
            hull = (jnp.maximum(px1, tx1) - jnp.minimum(px0, tx0)) * (
                jnp.maximum(py1, ty1) - jnp.minimum(py0, ty0))
            qq = (inter * hull + union * union) / (union * hull)

            l1 = (jnp.abs(px0 - tx0) + jnp.abs(py0 - ty0)
                  + jnp.abs(px1 - tx1) + jnp.abs(py1 - ty1))

            dx = ppx - tpx
            dy = ppy - tpy
            sq = dx * dx + dy * dy

            z = -lg
            cls2 = 2.0 * (jnp.maximum(z, 0.0)
                          + jnp.log1p(jnp.exp(-jnp.abs(z)))) + 2.0

            out_ref[i, pl.ds(s, bq), :] = (cls2 + 5.0 * l1 - 2.0 * qq
                                           + 0.25 * sq)

    def _cost_kernel(feat_ref, out_ref):
        g = pl.program_id(0)
        for i in range(ipb):
            _one_image(feat_ref, out_ref, g * ipb + i, i)

    return _cost_kernel


def kernel(pred_logits, pred_boxes, pred_positions, true_boxes,
           true_positions, query_batch_offsets, electron_batch_offsets):
    nb = query_batch_offsets.shape[0] - 1
    tq = pred_logits.shape[0]
    te = true_boxes.shape[0]
    q = tq // nb
    e = te // nb
    pad_q = jnp.zeros((tq, 1), jnp.float32)
    pad_e = jnp.zeros((te, 2), jnp.float32)
    feat = jnp.concatenate([
        jnp.concatenate([pred_boxes, pred_positions, pred_logits[:, None],
                         pad_q], axis=1),
        jnp.concatenate([true_boxes, true_positions, pad_e], axis=1),
    ], axis=0)
    return pl.pallas_call(
        _make_cost_kernel(q, e, tq, _IPB, _BQ),
        grid=(nb // _IPB,),
        in_specs=[pl.BlockSpec((tq + te, 8), lambda b: (0, 0))],
        out_specs=pl.BlockSpec((_IPB, q, e), lambda b: (b, 0, 0)),
        out_shape=jax.ShapeDtypeStruct((nb, q, e), jnp.float32),
    )(feat)


# speedup vs baseline: 1.1566x; 1.0056x over previous
import jax
import jax.numpy as jnp
from jax.experimental import pallas as pl
from jax.experimental.pallas import tpu as pltpu


_IPB = 2
_BQ = 256


def _make_cost_kernel(q, e, tq, ipb, bq):
    def _one_image(feat_ref, out_ref, b, i):
        es = pl.multiple_of(tq + b * e, e)
        tf = feat_ref[pl.ds(es, e), :].T
        tx0 = tf[0:1, :]
        ty0 = tf[1:2, :]
        tx1 = tf[2:3, :]
        ty1 = tf[3:4, :]
        tpx = tf[4:5, :]
        tpy = tf[5:6, :]
        area2 = (tx1 - tx0) * (ty1 - ty0)

        for s in range(0, q, bq):
            qs = pl.multiple_of(b * q + s, bq)
            pf = feat_ref[pl.ds(qs, bq), :]
            px0 = pf[:, 0:1]
            py0 = pf[:, 1:2]
            px1 = pf[:, 2:3]
            py1 = pf[:, 3:4]
            ppx = pf[:, 4:5]
            ppy = pf[:, 5:6]
            lg = pf[:, 6:7]

            area1 = (px1 - px0) * (py1 - py0)
            wx = jnp.maximum(
                jnp.minimum(px1, tx1) - jnp.maximum(px0, tx0), 0.0)
            wy = jnp.maximum(
                jnp.minimum(py1, ty1) - jnp.maximum(py0, ty0), 0.0)
            inter = wx * wy
            union = area1 + area2 - inter
            hull = (jnp.maximum(px1, tx1) - jnp.minimum(px0, tx0)) * (
                jnp.maximum(py1, ty1) - jnp.minimum(py0, ty0))
            qq = (inter * hull + union * union) / (union * hull)

            l1 = (jnp.abs(px0 - tx0) + jnp.abs(py0 - ty0)
                  + jnp.abs(px1 - tx1) + jnp.abs(py1 - ty1))

            dx = ppx - tpx
            dy = ppy - tpy
            sq = dx * dx + dy * dy

            z = -lg
            cls2 = 2.0 * (jnp.maximum(z, 0.0)
                          + jnp.log1p(jnp.exp(-jnp.abs(z)))) + 2.0

            out_ref[i, pl.ds(s, bq), :] = (cls2 + 5.0 * l1 - 2.0 * qq
                                           + 0.25 * sq)

    def _cost_kernel(feat_ref, out_ref):
        g = pl.program_id(0)
        for i in range(ipb):
            _one_image(feat_ref, out_ref, g * ipb + i, i)

    return _cost_kernel


def kernel(pred_logits, pred_boxes, pred_positions, true_boxes,
           true_positions, query_batch_offsets, electron_batch_offsets):
    nb = query_batch_offsets.shape[0] - 1
    tq = pred_logits.shape[0]
    te = true_boxes.shape[0]
    q = tq // nb
    e = te // nb
    pad_q = jnp.zeros((tq, 1), jnp.float32)
    pad_e = jnp.zeros((te, 2), jnp.float32)
    feat = jnp.concatenate([
        jnp.concatenate([pred_boxes, pred_positions, pred_logits[:, None],
                         pad_q], axis=1),
        jnp.concatenate([true_boxes, true_positions, pad_e], axis=1),
    ], axis=0)
    return pl.pallas_call(
        _make_cost_kernel(q, e, tq, _IPB, _BQ),
        grid=(nb // _IPB,),
        in_specs=[pl.BlockSpec((tq + te, 8), lambda b: (0, 0))],
        out_specs=pl.BlockSpec((_IPB, q, e), lambda b: (b, 0, 0)),
        out_shape=jax.ShapeDtypeStruct((nb, q, e), jnp.float32),
        compiler_params=pltpu.CompilerParams(
            dimension_semantics=("parallel",)),
    )(feat)


# speedup vs baseline: 1.1567x; 1.0001x over previous
import jax
import jax.numpy as jnp
from jax.experimental import pallas as pl
from jax.experimental.pallas import tpu as pltpu


_IPB = 2
_BQ = 256


def _make_cost_kernel(q, e, tq, ipb, bq):
    def _one_image(feat_ref, out_ref, b, i):
        es = pl.multiple_of(tq + b * e, e)
        tf = feat_ref[pl.ds(es, e), :].T
        tx0 = tf[0:1, :]
        ty0 = tf[1:2, :]
        tx1 = tf[2:3, :]
        ty1 = tf[3:4, :]
        tpx = tf[4:5, :]
        tpy = tf[5:6, :]
        area2 = (tx1 - tx0) * (ty1 - ty0)

        for s in range(0, q, bq):
            qs = pl.multiple_of(b * q + s, bq)
            pf = feat_ref[pl.ds(qs, bq), :]
            px0 = pf[:, 0:1]
            py0 = pf[:, 1:2]
            px1 = pf[:, 2:3]
            py1 = pf[:, 3:4]
            ppx = pf[:, 4:5]
            ppy = pf[:, 5:6]
            lg = pf[:, 6:7]

            area1 = (px1 - px0) * (py1 - py0)
            wx = jnp.maximum(
                jnp.minimum(px1, tx1) - jnp.maximum(px0, tx0), 0.0)
            wy = jnp.maximum(
                jnp.minimum(py1, ty1) - jnp.maximum(py0, ty0), 0.0)
            inter = wx * wy
            union = area1 + area2 - inter
            hull = (jnp.maximum(px1, tx1) - jnp.minimum(px0, tx0)) * (
                jnp.maximum(py1, ty1) - jnp.minimum(py0, ty0))
            qq = (inter * hull + union * union) / (union * hull)

            l1 = (jnp.abs(px0 - tx0) + jnp.abs(py0 - ty0)
                  + jnp.abs(px1 - tx1) + jnp.abs(py1 - ty1))

            dx = ppx - tpx
            dy = ppy - tpy
            sq = dx * dx + dy * dy

            z = -lg
            cls2 = 2.0 * (jnp.maximum(z, 0.0)
                          + jnp.log1p(jnp.exp(-jnp.abs(z)))) + 2.0

            out_ref[i, pl.ds(s, bq), :] = (cls2 + 5.0 * l1 - 2.0 * qq
                                           + 0.25 * sq)

    def _cost_kernel(feat_hbm, out_ref, feat_ref, sem):
        g = pl.program_id(0)

        @pl.when(g == 0)
        def _():
            cp = pltpu.make_async_copy(feat_hbm, feat_ref, sem)
            cp.start()
            cp.wait()

        for i in range(ipb):
            _one_image(feat_ref, out_ref, g * ipb + i, i)

    return _cost_kernel


def kernel(pred_logits, pred_boxes, pred_positions, true_boxes,
           true_positions, query_batch_offsets, electron_batch_offsets):
    nb = query_batch_offsets.shape[0] - 1
    tq = pred_logits.shape[0]
    te = true_boxes.shape[0]
    q = tq // nb
    e = te // nb
    pad_q = jnp.zeros((tq, 1), jnp.float32)
    pad_e = jnp.zeros((te, 2), jnp.float32)
    feat = jnp.concatenate([
        jnp.concatenate([pred_boxes, pred_positions, pred_logits[:, None],
                         pad_q], axis=1),
        jnp.concatenate([true_boxes, true_positions, pad_e], axis=1),
    ], axis=0)
    return pl.pallas_call(
        _make_cost_kernel(q, e, tq, _IPB, _BQ),
        grid=(nb // _IPB,),
        in_specs=[pl.BlockSpec(memory_space=pltpu.MemorySpace.HBM)],
        out_specs=pl.BlockSpec((_IPB, q, e), lambda b: (b, 0, 0)),
        out_shape=jax.ShapeDtypeStruct((nb, q, e), jnp.float32),
        compiler_params=pltpu.CompilerParams(
            dimension_semantics=("arbitrary",)),
        scratch_shapes=[pltpu.VMEM((tq + te, 8), jnp.float32),
                        pltpu.SemaphoreType.DMA],
    )(feat)
